# baseline (speedup 1.0000x reference)
import functools
import math

import jax
import jax.numpy as jnp
from jax.experimental import pallas as pl
from jax.experimental.pallas import tpu as pltpu

BLOCK = 32
MIN_K, MAX_K, ALPHA, THR = 4, 16, 0.1, 0.5
SUB = 4
ROWS = SUB * BLOCK
GB = 64
INV_SQRT_D = 1.0 / math.sqrt(64.0)


def _masks():
    i = jax.lax.broadcasted_iota(jnp.int32, (ROWS, ROWS), 0)
    j = jax.lax.broadcasted_iota(jnp.int32, (ROWS, ROWS), 1)
    blk = (i // BLOCK) == (j // BLOCK)
    tril = (i % BLOCK) >= (j % BLOCK)
    decay = 1.0 - ALPHA * jnp.abs(i % BLOCK - j % BLOCK).astype(jnp.float32) / BLOCK
    return blk, tril, decay


def _f_probs(qt, kt, blk, tril, decay):
    s = jax.lax.dot_general(qt, kt, (((1,), (1,)), ((), ())),
                            preferred_element_type=jnp.float32) * INV_SQRT_D
    f = jnp.where(blk & tril, jnp.maximum(s, 0.0) * decay, 0.0)
    fm = jnp.where(blk, f, -jnp.inf)
    m = jnp.max(fm, axis=-1, keepdims=True)
    e = jnp.where(blk, jnp.exp(f - m), 0.0)
    p = e / jnp.sum(e, axis=-1, keepdims=True)
    return f, p


def _ent_kernel(x_ref, wqT_ref, wkT_ref, out_ref):
    X = x_ref[...].reshape(GB * BLOCK, x_ref.shape[-1])
    q = jnp.dot(X, wqT_ref[...], preferred_element_type=jnp.float32)
    k = jnp.dot(X, wkT_ref[...], preferred_element_type=jnp.float32)
    blk, tril, decay = _masks()
    acc = jnp.zeros((), jnp.float32)
    for t in range(GB // SUB):
        rows = slice(t * ROWS, (t + 1) * ROWS)
        _, p = _f_probs(q[rows], k[rows], blk, tril, decay)
        acc = acc - jnp.sum(p * jnp.log(p + 1e-9))
    out_ref[...] = acc.reshape(1, 1, 1)


def _hybrid_kernel(x_ref, wqT_ref, wkT_ref, wvT_ref, p1b_ref, p2b_ref, out_ref):
    X = x_ref[...].reshape(GB * BLOCK, x_ref.shape[-1])
    q = jnp.dot(X, wqT_ref[...], preferred_element_type=jnp.float32)
    k = jnp.dot(X, wkT_ref[...], preferred_element_type=jnp.float32)
    v = jnp.dot(X, wvT_ref[...], preferred_element_type=jnp.float32)
    p1b = p1b_ref[...]
    p2b = p2b_ref[...]
    blk, tril, decay = _masks()
    for t in range(GB // SUB):
        rows = slice(t * ROWS, (t + 1) * ROWS)
        _, p = _f_probs(q[rows], k[rows], blk, tril, decay)
        lat = jnp.maximum(jnp.dot(p, p1b, preferred_element_type=jnp.float32), 0.0)
        lg = jnp.dot(lat, p2b, preferred_element_type=jnp.float32)
        lm = jnp.max(jnp.where(blk, lg, -jnp.inf), axis=-1, keepdims=True)
        e = jnp.where(blk, jnp.exp(lg - lm), 0.0)
        a = e / jnp.sum(e, axis=-1, keepdims=True)
        o = jnp.dot(a, v[rows], preferred_element_type=jnp.float32)
        out_ref[t * SUB:(t + 1) * SUB] = o.reshape(SUB, BLOCK, 64)


def _sparse_kernel(x_ref, wqT_ref, wkT_ref, wvT_ref, sc_ref, out_ref):
    kk = sc_ref[0, 0]
    amr = sc_ref[0, 1]
    gamma = sc_ref[0, 2]
    X = x_ref[...].reshape(GB * BLOCK, x_ref.shape[-1])
    q = jnp.dot(X, wqT_ref[...], preferred_element_type=jnp.float32)
    k = jnp.dot(X, wkT_ref[...], preferred_element_type=jnp.float32)
    v = jnp.dot(X, wvT_ref[...], preferred_element_type=jnp.float32)
    blk, tril, decay = _masks()
    rb = jax.lax.broadcasted_iota(jnp.int32, (ROWS, BLOCK), 0) // BLOCK
    jc = jax.lax.broadcasted_iota(jnp.int32, (ROWS, 1, BLOCK), 2)
    jr = jax.lax.broadcasted_iota(jnp.int32, (ROWS, BLOCK, 1), 1)
    for t in range(GB // SUB):
        rows = slice(t * ROWS, (t + 1) * ROWS)
        f, _ = _f_probs(q[rows], k[rows], blk, tril, decay)
        fc = jnp.zeros((ROWS, BLOCK), jnp.float32)
        for u in range(SUB):
            fc = fc + jnp.where(rb == u, f[:, u * BLOCK:(u + 1) * BLOCK], 0.0)
        mean = jnp.mean(fc, axis=-1, keepdims=True)
        var = jnp.sum((fc - mean) ** 2, axis=-1, keepdims=True) / (BLOCK - 1)
        sigma = jnp.sqrt(var)
        m = jnp.max(fc, axis=-1, keepdims=True)
        denom = jnp.maximum(m, sigma) + 1e-6
        nw = jnp.clip(jnp.floor(amr * fc / denom), 0.0, amr)
        fa = fc[:, None, :]
        fb = fc[:, :, None]
        cond = (fa > fb) | ((fa == fb) & (jc < jr))
        rank = jnp.sum(cond.astype(jnp.float32), axis=-1)
        sel = rank < jnp.minimum(kk, float(MAX_K))
        w = jnp.where(sel, nw, 0.0) / gamma
        wb = jnp.where(blk, jnp.concatenate([w] * SUB, axis=1), 0.0)
        o = jnp.dot(wb, v[rows], preferred_element_type=jnp.float32)
        out_ref[t * SUB:(t + 1) * SUB] = o.reshape(SUB, BLOCK, 64)


def _full_spec(ndim):
    return pl.BlockSpec(None, lambda i: (0,) * ndim)


def kernel(x, Wk, Wq, Wv, P1, P2, gamma):
    B, T, C = x.shape
    nsteps = B // GB
    wqT = Wq.T
    wkT = Wk.T
    wvT = Wv.T
    eye = jnp.eye(SUB, dtype=jnp.float32)
    p1b = jnp.kron(eye, P1.T)
    p2b = jnp.kron(eye, P2.T)

    w_spec = pl.BlockSpec((C, 64), lambda i: (0, 0))
    x_spec = pl.BlockSpec((GB, T, C), lambda i: (i, 0, 0))
    out_spec = pl.BlockSpec((GB, T, 64), lambda i: (i, 0, 0))

    ent = pl.pallas_call(
        _ent_kernel,
        grid=(nsteps,),
        in_specs=[x_spec, w_spec, w_spec],
        out_specs=pl.BlockSpec((1, 1, 1), lambda i: (i, 0, 0)),
        out_shape=jax.ShapeDtypeStruct((nsteps, 1, 1), jnp.float32),
    )(x, wqT, wkT)

    a = jnp.sum(ent) / (B * T * math.log(T))
    hybrid = a > THR
    kk = jnp.clip(jnp.floor(MIN_K + (MAX_K - MIN_K) * a), MIN_K, MAX_K)
    amr = jnp.floor(15 + (127 - 15) * a)

    def _hybrid_branch(ops):
        x, wqT, wkT, wvT, p1b, p2b, _ = ops
        return pl.pallas_call(
            _hybrid_kernel,
            grid=(nsteps,),
            in_specs=[x_spec, w_spec, w_spec, w_spec,
                      pl.BlockSpec((ROWS, ROWS), lambda i: (0, 0)),
                      pl.BlockSpec((ROWS, ROWS), lambda i: (0, 0))],
            out_specs=out_spec,
            out_shape=jax.ShapeDtypeStruct((B, T, 64), jnp.float32),
        )(x, wqT, wkT, wvT, p1b, p2b)

    def _sparse_branch(ops):
        x, wqT, wkT, wvT, _, _, sc = ops
        return pl.pallas_call(
            _sparse_kernel,
            grid=(nsteps,),
            in_specs=[x_spec, w_spec, w_spec, w_spec,
                      pl.BlockSpec(memory_space=pltpu.SMEM)],
            out_specs=out_spec,
            out_shape=jax.ShapeDtypeStruct((B, T, 64), jnp.float32),
        )(x, wqT, wkT, wvT, sc)

    sc = jnp.stack([kk, amr, gamma.astype(jnp.float32)]).reshape(1, 3)
    ops = (x, wqT, wkT, wvT, p1b, p2b, sc)
    return jax.lax.cond(hybrid, _hybrid_branch, _sparse_branch, ops)


# speedup vs baseline: 1.9035x; 1.9035x over previous
import functools
import math

import jax
import jax.numpy as jnp
from jax.experimental import pallas as pl
from jax.experimental.pallas import tpu as pltpu

BLOCK = 32
MIN_K, MAX_K, ALPHA, THR = 4, 16, 0.1, 0.5
SUB = 4
ROWS = SUB * BLOCK
GB = 64
INV_SQRT_D = 1.0 / math.sqrt(64.0)


def _masks_tall(nrows):
    i = jax.lax.broadcasted_iota(jnp.int32, (nrows, ROWS), 0)
    j = jax.lax.broadcasted_iota(jnp.int32, (nrows, ROWS), 1)
    blk = ((i // BLOCK) % SUB) == (j // BLOCK)
    tril = (i % BLOCK) >= (j % BLOCK)
    decay = 1.0 - ALPHA * jnp.abs(i % BLOCK - j % BLOCK).astype(jnp.float32) / BLOCK
    return blk, tril, decay


def _scores_all(q, k):
    nt = q.shape[0] // ROWS
    outs = []
    for t in range(nt):
        rows = slice(t * ROWS, (t + 1) * ROWS)
        outs.append(jax.lax.dot_general(q[rows], k[rows], (((1,), (1,)), ((), ())),
                                        preferred_element_type=jnp.float32))
    return jnp.concatenate(outs, axis=0) * INV_SQRT_D


def _probs_all(s, blk, tril, decay):
    f = jnp.where(blk & tril, jnp.maximum(s, 0.0) * decay, 0.0)
    fm = jnp.where(blk, f, -jnp.inf)
    m = jnp.max(fm, axis=-1, keepdims=True)
    e = jnp.where(blk, jnp.exp(f - m), 0.0)
    p = e / jnp.sum(e, axis=-1, keepdims=True)
    return f, p


def _ent_kernel(x_ref, wqT_ref, wkT_ref, out_ref):
    X = x_ref[...].reshape(GB * BLOCK, x_ref.shape[-1])
    q = jnp.dot(X, wqT_ref[...], preferred_element_type=jnp.float32)
    k = jnp.dot(X, wkT_ref[...], preferred_element_type=jnp.float32)
    blk, tril, decay = _masks_tall(GB * BLOCK)
    s = _scores_all(q, k)
    _, p = _probs_all(s, blk, tril, decay)
    acc = -jnp.sum(p * jnp.log(p + 1e-9))
    out_ref[...] = acc.reshape(1, 1, 1)


def _hybrid_kernel(x_ref, wqT_ref, wkT_ref, wvT_ref, p1b_ref, p2b_ref, out_ref):
    X = x_ref[...].reshape(GB * BLOCK, x_ref.shape[-1])
    q = jnp.dot(X, wqT_ref[...], preferred_element_type=jnp.float32)
    k = jnp.dot(X, wkT_ref[...], preferred_element_type=jnp.float32)
    v = jnp.dot(X, wvT_ref[...], preferred_element_type=jnp.float32)
    blk, tril, decay = _masks_tall(GB * BLOCK)
    s = _scores_all(q, k)
    _, p = _probs_all(s, blk, tril, decay)
    lat = jnp.maximum(jnp.dot(p, p1b_ref[...], preferred_element_type=jnp.float32), 0.0)
    lg = jnp.dot(lat, p2b_ref[...], preferred_element_type=jnp.float32)
    lm = jnp.max(jnp.where(blk, lg, -jnp.inf), axis=-1, keepdims=True)
    e = jnp.where(blk, jnp.exp(lg - lm), 0.0)
    a = e / jnp.sum(e, axis=-1, keepdims=True)
    for t in range(GB // SUB):
        rows = slice(t * ROWS, (t + 1) * ROWS)
        o = jnp.dot(a[rows], v[rows], preferred_element_type=jnp.float32)
        out_ref[t * SUB:(t + 1) * SUB] = o.reshape(SUB, BLOCK, 64)


def _sparse_kernel(x_ref, wqT_ref, wkT_ref, wvT_ref, sc_ref, out_ref):
    kk = sc_ref[0, 0]
    amr = sc_ref[0, 1]
    gamma = sc_ref[0, 2]
    nr = GB * BLOCK
    X = x_ref[...].reshape(nr, x_ref.shape[-1])
    q = jnp.dot(X, wqT_ref[...], preferred_element_type=jnp.float32)
    k = jnp.dot(X, wkT_ref[...], preferred_element_type=jnp.float32)
    v = jnp.dot(X, wvT_ref[...], preferred_element_type=jnp.float32)
    blk, tril, decay = _masks_tall(nr)
    s = _scores_all(q, k)
    f, _ = _probs_all(s, blk, tril, decay)
    rb = (jax.lax.broadcasted_iota(jnp.int32, (nr, BLOCK), 0) // BLOCK) % SUB
    fc = jnp.zeros((nr, BLOCK), jnp.float32)
    for u in range(SUB):
        fc = fc + jnp.where(rb == u, f[:, u * BLOCK:(u + 1) * BLOCK], 0.0)
    mean = jnp.mean(fc, axis=-1, keepdims=True)
    var = jnp.sum((fc - mean) ** 2, axis=-1, keepdims=True) / (BLOCK - 1)
    sigma = jnp.sqrt(var)
    m = jnp.max(fc, axis=-1, keepdims=True)
    denom = jnp.maximum(m, sigma) + 1e-6
    nw = jnp.clip(jnp.floor(amr * fc / denom), 0.0, amr)
    col = jax.lax.broadcasted_iota(jnp.int32, (nr, BLOCK), 1)
    rank = jnp.zeros((nr, BLOCK), jnp.float32)
    for sft in range(1, BLOCK):
        fs = jnp.concatenate([fc[:, sft:], fc[:, :sft]], axis=1)
        cond = (fs > fc) | ((fs == fc) & (col + sft >= BLOCK))
        rank = rank + cond.astype(jnp.float32)
    sel = rank < jnp.minimum(kk, float(MAX_K))
    w = jnp.where(sel, nw, 0.0) / gamma
    wb = jnp.where(blk, jnp.concatenate([w] * SUB, axis=1), 0.0)
    for t in range(GB // SUB):
        rows = slice(t * ROWS, (t + 1) * ROWS)
        o = jnp.dot(wb[rows], v[rows], preferred_element_type=jnp.float32)
        out_ref[t * SUB:(t + 1) * SUB] = o.reshape(SUB, BLOCK, 64)


def _full_spec(ndim):
    return pl.BlockSpec(None, lambda i: (0,) * ndim)


def kernel(x, Wk, Wq, Wv, P1, P2, gamma):
    B, T, C = x.shape
    nsteps = B // GB
    wqT = Wq.T
    wkT = Wk.T
    wvT = Wv.T
    eye = jnp.eye(SUB, dtype=jnp.float32)
    p1b = jnp.kron(eye, P1.T)
    p2b = jnp.kron(eye, P2.T)

    w_spec = pl.BlockSpec((C, 64), lambda i: (0, 0))
    x_spec = pl.BlockSpec((GB, T, C), lambda i: (i, 0, 0))
    out_spec = pl.BlockSpec((GB, T, 64), lambda i: (i, 0, 0))

    ent = pl.pallas_call(
        _ent_kernel,
        grid=(nsteps,),
        in_specs=[x_spec, w_spec, w_spec],
        out_specs=pl.BlockSpec((1, 1, 1), lambda i: (i, 0, 0)),
        out_shape=jax.ShapeDtypeStruct((nsteps, 1, 1), jnp.float32),
    )(x, wqT, wkT)

    a = jnp.sum(ent) / (B * T * math.log(T))
    hybrid = a > THR
    kk = jnp.clip(jnp.floor(MIN_K + (MAX_K - MIN_K) * a), MIN_K, MAX_K)
    amr = jnp.floor(15 + (127 - 15) * a)

    def _hybrid_branch(ops):
        x, wqT, wkT, wvT, p1b, p2b, _ = ops
        return pl.pallas_call(
            _hybrid_kernel,
            grid=(nsteps,),
            in_specs=[x_spec, w_spec, w_spec, w_spec,
                      pl.BlockSpec((ROWS, ROWS), lambda i: (0, 0)),
                      pl.BlockSpec((ROWS, ROWS), lambda i: (0, 0))],
            out_specs=out_spec,
            out_shape=jax.ShapeDtypeStruct((B, T, 64), jnp.float32),
        )(x, wqT, wkT, wvT, p1b, p2b)

    def _sparse_branch(ops):
        x, wqT, wkT, wvT, _, _, sc = ops
        return pl.pallas_call(
            _sparse_kernel,
            grid=(nsteps,),
            in_specs=[x_spec, w_spec, w_spec, w_spec,
                      pl.BlockSpec(memory_space=pltpu.SMEM)],
            out_specs=out_spec,
            out_shape=jax.ShapeDtypeStruct((B, T, 64), jnp.float32),
        )(x, wqT, wkT, wvT, sc)

    sc = jnp.stack([kk, amr, gamma.astype(jnp.float32)]).reshape(1, 3)
    ops = (x, wqT, wkT, wvT, p1b, p2b, sc)
    return jax.lax.cond(hybrid, _hybrid_branch, _sparse_branch, ops)
